# baseline (speedup 1.0000x reference)
import dataclasses

import jax
import jax.numpy as jnp
from jax.experimental import pallas as pl
from jax.experimental.pallas import tpu as pltpu
from jax.experimental.pallas import tpu_sc as plsc

_E = 64
_D = 48
_TM = 512
_RB = 8


def _router_kernel(x_ref, wr_ref, o_ref):
    xb = x_ref[...].astype(jnp.bfloat16)
    o_ref[...] = jax.lax.dot_general(
        xb, wr_ref[...].astype(jnp.bfloat16), (((1,), (1,)), ((), ())),
        preferred_element_type=jnp.float32)


def _sc_argmax(logits):
    n = logits.shape[0]
    mesh = plsc.VectorSubcoreMesh(core_axis_name="c", subcore_axis_name="s")

    cp = pltpu.CompilerParams()
    if "needs_layout_passes" in pltpu.CompilerParams.__dataclass_fields__:
        cp = dataclasses.replace(cp, needs_layout_passes=False)

    @pl.kernel(out_type=jax.ShapeDtypeStruct((n, 16), jnp.int32), mesh=mesh,
               compiler_params=cp)
    def _argmax_kernel(l_hbm, o_hbm):
        def body(l_vmem, o_vmem):
            @pl.loop(0, _RB)
            def _(r):
                v0 = l_vmem.at[r, pl.ds(0, 16)][...]
                v1 = l_vmem.at[r, pl.ds(16, 16)][...]
                v2 = l_vmem.at[r, pl.ds(32, 16)][...]
                v3 = l_vmem.at[r, pl.ds(48, 16)][...]
                m = jnp.maximum(jnp.maximum(v0, v1), jnp.maximum(v2, v3))
                mr = jax.lax.reduce_max(m, (0,))
                idx = jax.lax.iota(jnp.int32, 16)
                big = jnp.full((16,), _E, jnp.int32)
                c0 = jnp.where(v0 == mr, idx, big)
                c1 = jnp.where(v1 == mr, idx + 16, big)
                c2 = jnp.where(v2 == mr, idx + 32, big)
                c3 = jnp.where(v3 == mr, idx + 48, big)
                cm = jnp.minimum(jnp.minimum(c0, c1), jnp.minimum(c2, c3))
                eid = jax.lax.reduce_min(cm, (0,))
                o_vmem.at[r, :][...] = jnp.broadcast_to(eid, (16,))

        pltpu.emit_pipeline(
            body,
            grid=(n // _RB,),
            in_specs=[pl.BlockSpec((_RB, _E), index_map=lambda i: (i, 0))],
            out_specs=[pl.BlockSpec((_RB, 16), index_map=lambda i: (i, 0))],
            core_axis_name=("c", "s"),
            dimension_semantics=(pltpu.PARALLEL,),
        )(l_hbm, o_hbm)

    return _argmax_kernel(logits)


def _moe_dense_kernel(x_ref, eid_ref, w1_ref, w2_ref, o_ref):
    xb = x_ref[...].astype(jnp.bfloat16)
    lo = eid_ref[...][:, :1] * _D

    h = jax.lax.dot_general(
        xb, w1_ref[...], (((1,), (0,)), ((), ())),
        preferred_element_type=jnp.float32)
    h = jnp.maximum(h, 0.0)
    h = h * h
    col = jax.lax.broadcasted_iota(jnp.int32, h.shape, 1)
    rel = (col - lo).astype(jnp.uint32)
    g = jnp.where(rel < _D, h, 0.0).astype(jnp.bfloat16)

    o_ref[...] = jax.lax.dot_general(
        g, w2_ref[...].astype(jnp.bfloat16), (((1,), (0,)), ((), ())),
        preferred_element_type=jnp.float32)


def kernel(x, Wr, W1, W2):
    B, T, C = x.shape
    N = B * T
    E, _, D = W1.shape
    x_flat = x.reshape(N, C)
    w1cat = W1.astype(jnp.bfloat16).transpose(1, 0, 2).reshape(C, E * D)
    w2cat = W2.reshape(E * D, C)

    logits = pl.pallas_call(
        _router_kernel,
        out_shape=jax.ShapeDtypeStruct((N, E), jnp.float32),
    )(x_flat, Wr)

    eid = _sc_argmax(logits)

    out = pl.pallas_call(
        _moe_dense_kernel,
        grid=(N // _TM,),
        in_specs=[
            pl.BlockSpec((_TM, C), lambda i: (i, 0)),
            pl.BlockSpec((_TM, 16), lambda i: (i, 0)),
            pl.BlockSpec((C, E * D), lambda i: (0, 0)),
            pl.BlockSpec((E * D, C), lambda i: (0, 0)),
        ],
        out_specs=pl.BlockSpec((_TM, C), lambda i: (i, 0)),
        out_shape=jax.ShapeDtypeStruct((N, C), jnp.float32),
        compiler_params=pltpu.CompilerParams(
            dimension_semantics=("parallel",)),
    )(x_flat, eid, w1cat, w2cat)
    return out.reshape(B, T, C)


# speedup vs baseline: 1.5109x; 1.5109x over previous
import jax
import jax.numpy as jnp
from jax.experimental import pallas as pl
from jax.experimental.pallas import tpu as pltpu

_E = 64
_D = 48
_TM = 512


def _moe_dense_kernel(x_ref, wr_ref, w1_ref, w2_ref, o_ref):
    xb = x_ref[...].astype(jnp.bfloat16)
    logits = jax.lax.dot_general(
        xb, wr_ref[...].astype(jnp.bfloat16), (((1,), (1,)), ((), ())),
        preferred_element_type=jnp.float32)
    m = jnp.max(logits, axis=-1, keepdims=True)
    lane = jax.lax.broadcasted_iota(jnp.int32, logits.shape, 1)
    eid = jnp.min(jnp.where(logits == m, lane, _E), axis=-1)

    h = jax.lax.dot_general(
        xb, w1_ref[...], (((1,), (0,)), ((), ())),
        preferred_element_type=jnp.float32)
    h = jnp.maximum(h, 0.0)
    h = h * h
    col = jax.lax.broadcasted_iota(jnp.int32, h.shape, 1)
    rel = (col - (eid * _D)[:, None]).astype(jnp.uint32)
    g = jnp.where(rel < _D, h, 0.0).astype(jnp.bfloat16)

    o_ref[...] = jax.lax.dot_general(
        g, w2_ref[...].astype(jnp.bfloat16), (((1,), (0,)), ((), ())),
        preferred_element_type=jnp.float32)


def kernel(x, Wr, W1, W2):
    B, T, C = x.shape
    N = B * T
    E, _, D = W1.shape
    x_flat = x.reshape(N, C)
    w1cat = W1.astype(jnp.bfloat16).transpose(1, 0, 2).reshape(C, E * D)
    w2cat = W2.reshape(E * D, C)

    out = pl.pallas_call(
        _moe_dense_kernel,
        grid=(N // _TM,),
        in_specs=[
            pl.BlockSpec((_TM, C), lambda i: (i, 0)),
            pl.BlockSpec((E, C), lambda i: (0, 0)),
            pl.BlockSpec((C, E * D), lambda i: (0, 0)),
            pl.BlockSpec((E * D, C), lambda i: (0, 0)),
        ],
        out_specs=pl.BlockSpec((_TM, C), lambda i: (i, 0)),
        out_shape=jax.ShapeDtypeStruct((N, C), jnp.float32),
        compiler_params=pltpu.CompilerParams(
            dimension_semantics=("parallel",)),
    )(x_flat, Wr, w1cat, w2cat)
    return out.reshape(B, T, C)
